# baseline (speedup 1.0000x reference)
import functools

import jax
import jax.numpy as jnp
from jax.experimental import pallas as pl
from jax.experimental.pallas import tpu as pltpu


def _fused_gcn_kernel(x_ref, w1_ref, w2_ref, b1_ref, b2_ref, adj_hbm,
                      out_ref, adj_vmem, s1_ref, s2_ref, copy_sems,
                      *, tm, num_classes):
    phase = pl.program_id(0)
    i = pl.program_id(1)
    row0 = pl.multiple_of(i * tm, tm)

    @pl.when(phase == 0)
    def _():
        pltpu.make_async_copy(
            adj_hbm.at[pl.ds(row0, tm), :],
            adj_vmem.at[pl.ds(row0, tm), :],
            copy_sems.at[i],
        ).start()
        s1_ref[pl.ds(row0, tm), :] = jnp.dot(
            x_ref[...], w1_ref[...], preferred_element_type=jnp.float32
        ).astype(s1_ref.dtype)

    @pl.when(phase == 1)
    def _():
        pltpu.make_async_copy(
            adj_hbm.at[pl.ds(row0, tm), :],
            adj_vmem.at[pl.ds(row0, tm), :],
            copy_sems.at[i],
        ).wait()
        ht = jax.lax.dot_general(
            s1_ref[...], adj_vmem[pl.ds(row0, tm), :],
            (((0,), (1,)), ((), ())),
            preferred_element_type=jnp.float32)
        ht = jnp.maximum(ht + b1_ref[...].reshape(-1, 1), 0.0)
        s2_ref[pl.ds(row0, tm), :] = jax.lax.dot_general(
            ht.astype(jnp.bfloat16), w2_ref[...],
            (((0,), (0,)), ((), ())),
            preferred_element_type=jnp.float32
        ).astype(s2_ref.dtype)

    @pl.when(phase == 2)
    def _():
        zt = jax.lax.dot_general(
            s2_ref[...], adj_vmem[pl.ds(row0, tm), :],
            (((0,), (1,)), ((), ())),
            preferred_element_type=jnp.float32)
        z = zt.T + b2_ref[...]
        lane = jax.lax.broadcasted_iota(jnp.int32, z.shape, 1)
        z = jnp.where(lane < num_classes, z, jnp.float32(-1e30))
        m = jnp.max(z, axis=1, keepdims=True)
        shifted = z - m
        lse = jnp.log(jnp.sum(jnp.exp(shifted), axis=1, keepdims=True))
        out_ref[...] = (shifted - lse).astype(out_ref.dtype)


@functools.partial(jax.jit, static_argnames=("n_nodes", "num_classes", "tm"))
def _gcn_forward(x_p, adj_p, w1_p, b1_p, w2_p, b2_p, *, n_nodes, num_classes,
                 tm):
    N_pad, F_pad = x_p.shape
    H_pad = w1_p.shape[1]
    C_pad = w2_p.shape[1]
    nr = N_pad // tm

    out_p = pl.pallas_call(
        functools.partial(_fused_gcn_kernel, tm=tm, num_classes=num_classes),
        out_shape=jax.ShapeDtypeStruct((N_pad, C_pad), jnp.float32),
        grid=(3, nr),
        in_specs=[
            pl.BlockSpec((tm, F_pad),
                         lambda p, i: (jnp.where(p == 0, i, nr - 1), 0)),
            pl.BlockSpec((F_pad, H_pad), lambda p, i: (0, 0)),
            pl.BlockSpec((H_pad, C_pad), lambda p, i: (0, 0)),
            pl.BlockSpec((1, H_pad), lambda p, i: (0, 0)),
            pl.BlockSpec((1, C_pad), lambda p, i: (0, 0)),
            pl.BlockSpec(memory_space=pl.ANY),
        ],
        out_specs=pl.BlockSpec((tm, C_pad),
                               lambda p, i: (jnp.where(p == 2, i, 0), 0)),
        scratch_shapes=[
            pltpu.VMEM((N_pad, N_pad), jnp.bfloat16),
            pltpu.VMEM((N_pad, H_pad), jnp.bfloat16),
            pltpu.VMEM((N_pad, C_pad), jnp.bfloat16),
            pltpu.SemaphoreType.DMA((nr,)),
        ],
        compiler_params=pltpu.CompilerParams(
            dimension_semantics=("arbitrary", "arbitrary"),
            vmem_limit_bytes=56 << 20,
        ),
    )(x_p, w1_p, w2_p, b1_p, b2_p, adj_p)

    return out_p[:n_nodes, :num_classes]


def kernel(x_p, adj_p, w1_p, b1_p, w2_p, b2_p):
    return _gcn_forward(x_p, adj_p, w1_p, b1_p, w2_p, b2_p,
                        n_nodes=4096, num_classes=7, tm=256)


# speedup vs baseline: 1.3467x; 1.3467x over previous
import functools

import jax
import jax.numpy as jnp
from jax.experimental import pallas as pl
from jax.experimental.pallas import tpu as pltpu


def _fused_gcn_kernel(x_ref, w1_ref, w2_ref, b1_ref, b2_ref, adj_hbm,
                      out_ref, adj_vmem, s1_ref, s2_ref, copy_sems,
                      *, tm, num_classes):
    phase = pl.program_id(0)
    i = pl.program_id(1)
    row0 = pl.multiple_of(i * tm, tm)

    @pl.when(phase == 0)
    def _():
        pltpu.make_async_copy(
            adj_hbm.at[pl.ds(row0, tm), :],
            adj_vmem.at[pl.ds(row0, tm), :],
            copy_sems.at[i],
        ).start()
        s1_ref[pl.ds(row0, tm), :] = jnp.dot(
            x_ref[...], w1_ref[...], preferred_element_type=jnp.float32
        ).astype(s1_ref.dtype)

    @pl.when(phase == 1)
    def _():
        pltpu.make_async_copy(
            adj_hbm.at[pl.ds(row0, tm), :],
            adj_vmem.at[pl.ds(row0, tm), :],
            copy_sems.at[i],
        ).wait()
        h = jnp.dot(adj_vmem[pl.ds(row0, tm), :], s1_ref[...],
                    preferred_element_type=jnp.float32)
        h = jnp.maximum(h + b1_ref[...], 0.0)
        s2_ref[pl.ds(row0, tm), :] = jnp.dot(
            h.astype(jnp.bfloat16), w2_ref[...],
            preferred_element_type=jnp.float32
        ).astype(s2_ref.dtype)

    @pl.when(phase == 2)
    def _():
        z = jnp.dot(adj_vmem[pl.ds(row0, tm), :], s2_ref[...],
                    preferred_element_type=jnp.float32)
        z = z + b2_ref[...]
        lane = jax.lax.broadcasted_iota(jnp.int32, z.shape, 1)
        z = jnp.where(lane < num_classes, z, jnp.float32(-1e30))
        m = jnp.max(z, axis=1, keepdims=True)
        shifted = z - m
        lse = jnp.log(jnp.sum(jnp.exp(shifted), axis=1, keepdims=True))
        out_ref[...] = (shifted - lse).astype(out_ref.dtype)


@functools.partial(jax.jit, static_argnames=("n_nodes", "num_classes", "tm"))
def _gcn_forward(x_p, adj_p, w1_p, b1_p, w2_p, b2_p, *, n_nodes, num_classes,
                 tm):
    N_pad, F_pad = x_p.shape
    H_pad = w1_p.shape[1]
    C_pad = w2_p.shape[1]
    nr = N_pad // tm

    out_p = pl.pallas_call(
        functools.partial(_fused_gcn_kernel, tm=tm, num_classes=num_classes),
        out_shape=jax.ShapeDtypeStruct((N_pad, C_pad), jnp.float32),
        grid=(3, nr),
        in_specs=[
            pl.BlockSpec((tm, F_pad),
                         lambda p, i: (jnp.where(p == 0, i, nr - 1), 0)),
            pl.BlockSpec((F_pad, H_pad), lambda p, i: (0, 0)),
            pl.BlockSpec((H_pad, C_pad), lambda p, i: (0, 0)),
            pl.BlockSpec((1, H_pad), lambda p, i: (0, 0)),
            pl.BlockSpec((1, C_pad), lambda p, i: (0, 0)),
            pl.BlockSpec(memory_space=pl.ANY),
        ],
        out_specs=pl.BlockSpec((tm, C_pad),
                               lambda p, i: (jnp.where(p == 2, i, 0), 0)),
        scratch_shapes=[
            pltpu.VMEM((N_pad, N_pad), jnp.bfloat16),
            pltpu.VMEM((N_pad, H_pad), jnp.bfloat16),
            pltpu.VMEM((N_pad, C_pad), jnp.bfloat16),
            pltpu.SemaphoreType.DMA((nr,)),
        ],
        compiler_params=pltpu.CompilerParams(
            dimension_semantics=("arbitrary", "arbitrary"),
            vmem_limit_bytes=56 << 20,
        ),
    )(x_p, w1_p, w2_p, b1_p, b2_p, adj_p)

    return out_p[:n_nodes, :num_classes]


def kernel(x_p, adj_p, w1_p, b1_p, w2_p, b2_p):
    return _gcn_forward(x_p, adj_p, w1_p, b1_p, w2_p, b2_p,
                        n_nodes=4096, num_classes=7, tm=512)


# speedup vs baseline: 1.5295x; 1.1357x over previous
import functools

import jax
import jax.numpy as jnp
from jax.experimental import pallas as pl
from jax.experimental.pallas import tpu as pltpu


def _fused_gcn_kernel(x_ref, w1_ref, w2_ref, b1_ref, b2_ref, adj_hbm,
                      out_ref, adj_vmem, s1_ref, s2_ref, copy_sems,
                      *, tm, num_classes):
    phase = pl.program_id(0)
    i = pl.program_id(1)
    row0 = pl.multiple_of(i * tm, tm)

    @pl.when(phase == 0)
    def _():
        pltpu.make_async_copy(
            adj_hbm.at[pl.ds(row0, tm), :],
            adj_vmem.at[pl.ds(row0, tm), :],
            copy_sems.at[i],
        ).start()
        s1_ref[pl.ds(row0, tm), :] = jnp.dot(
            x_ref[...], w1_ref[...], preferred_element_type=jnp.float32
        ).astype(s1_ref.dtype)

    @pl.when(phase == 1)
    def _():
        pltpu.make_async_copy(
            adj_hbm.at[pl.ds(row0, tm), :],
            adj_vmem.at[pl.ds(row0, tm), :],
            copy_sems.at[i],
        ).wait()
        h = jnp.dot(adj_vmem[pl.ds(row0, tm), :], s1_ref[...],
                    preferred_element_type=jnp.float32)
        h = jnp.maximum(h + b1_ref[...], 0.0)
        s2_ref[pl.ds(row0, tm), :] = jnp.dot(
            h.astype(jnp.bfloat16), w2_ref[...],
            preferred_element_type=jnp.float32
        ).astype(s2_ref.dtype)

    @pl.when(phase == 2)
    def _():
        z = jnp.dot(adj_vmem[pl.ds(row0, tm), :], s2_ref[...],
                    preferred_element_type=jnp.float32)
        z = z + b2_ref[...]
        lane = jax.lax.broadcasted_iota(jnp.int32, z.shape, 1)
        z = jnp.where(lane < num_classes, z, jnp.float32(-1e30))
        m = jnp.max(z, axis=1, keepdims=True)
        shifted = z - m
        lse = jnp.log(jnp.sum(jnp.exp(shifted), axis=1, keepdims=True))
        out_ref[...] = (shifted - lse).astype(out_ref.dtype)


@functools.partial(jax.jit, static_argnames=("n_nodes", "num_classes", "tm"))
def _gcn_forward(x_p, adj_p, w1_p, b1_p, w2_p, b2_p, *, n_nodes, num_classes,
                 tm):
    N_pad, F_pad = x_p.shape
    H_pad = w1_p.shape[1]
    C_pad = w2_p.shape[1]
    nr = N_pad // tm

    out_p = pl.pallas_call(
        functools.partial(_fused_gcn_kernel, tm=tm, num_classes=num_classes),
        out_shape=jax.ShapeDtypeStruct((N_pad, C_pad), jnp.float32),
        grid=(3, nr),
        in_specs=[
            pl.BlockSpec((tm, F_pad),
                         lambda p, i: (jnp.where(p == 0, i, nr - 1), 0)),
            pl.BlockSpec((F_pad, H_pad), lambda p, i: (0, 0)),
            pl.BlockSpec((H_pad, C_pad), lambda p, i: (0, 0)),
            pl.BlockSpec((1, H_pad), lambda p, i: (0, 0)),
            pl.BlockSpec((1, C_pad), lambda p, i: (0, 0)),
            pl.BlockSpec(memory_space=pl.ANY),
        ],
        out_specs=pl.BlockSpec((tm, C_pad),
                               lambda p, i: (jnp.where(p == 2, i, 0), 0)),
        scratch_shapes=[
            pltpu.VMEM((N_pad, N_pad), jnp.bfloat16),
            pltpu.VMEM((N_pad, H_pad), jnp.bfloat16),
            pltpu.VMEM((N_pad, C_pad), jnp.bfloat16),
            pltpu.SemaphoreType.DMA((nr,)),
        ],
        compiler_params=pltpu.CompilerParams(
            dimension_semantics=("arbitrary", "arbitrary"),
            vmem_limit_bytes=56 << 20,
        ),
    )(x_p, w1_p, w2_p, b1_p, b2_p, adj_p)

    return out_p[:n_nodes, :num_classes]


def kernel(x_p, adj_p, w1_p, b1_p, w2_p, b2_p):
    return _gcn_forward(x_p, adj_p, w1_p, b1_p, w2_p, b2_p,
                        n_nodes=4096, num_classes=7, tm=1024)
